# baseline (speedup 1.0000x reference)
import functools

import jax
import jax.numpy as jnp
from jax import lax
from jax.experimental import pallas as pl
from jax.experimental.pallas import tpu as pltpu
from jax.experimental.pallas import tpu_sc as plsc

N = 10000
E = 320000
D = 128

NC = 2
NS = 16
NW = NC * NS

NPAD = 10240
RPT = NPAD // NS
EPW = E // NW

K = 40
NCH = EPW // K
NCHP = 256
QC = 64
NWAVE = NCH // 2

KD = 80
DCH = EPW // KD
DSG = 8
DFULL = DCH // DSG

_mesh = plsc.VectorSubcoreMesh(core_axis_name="c", subcore_axis_name="s")


@functools.partial(
    pl.kernel,
    mesh=_mesh,
    out_type=[
        jax.ShapeDtypeStruct((NC, NPAD), jnp.float32),
        jax.ShapeDtypeStruct((NC, NPAD), jnp.float32),
    ],
    scratch_types=[
        pltpu.VMEM((DCH + 3, KD), jnp.int32),
        pltpu.VMEM((DCH + 3, KD), jnp.int32),
        pltpu.VMEM((KD,), jnp.float32),
        pltpu.VMEM_SHARED((NPAD,), jnp.float32),
        pltpu.VMEM_SHARED((NPAD,), jnp.float32),
        pltpu.SemaphoreType.DMA,
    ],
)
def _deg_kernel(srcd_hbm, dstd_hbm, zeros1_hbm, out_o, out_i,
                src_w, dst_w, ones_v, ho, hi, sem):
    c = lax.axis_index("c")
    s = lax.axis_index("s")
    w = c * NS + s
    r0 = s * RPT

    pltpu.sync_copy(zeros1_hbm.at[pl.ds(r0, RPT)], ho.at[pl.ds(r0, RPT)])
    pltpu.sync_copy(zeros1_hbm.at[pl.ds(r0, RPT)], hi.at[pl.ds(r0, RPT)])
    pltpu.sync_copy(srcd_hbm.at[w], src_w)
    pltpu.sync_copy(dstd_hbm.at[w], dst_w)
    for j in range(KD // 16):
        ones_v[pl.ds(j * 16, 16)] = jnp.full((16,), 1.0, jnp.float32)
    plsc.subcore_barrier()

    def burst(j0, count):
        handles = []
        for b in range(count):
            handles.append(pltpu.async_copy(
                ones_v, ho.at[src_w.at[j0 + b]], sem, add=True))
            handles.append(pltpu.async_copy(
                ones_v, hi.at[dst_w.at[j0 + b]], sem, add=True))
        for h in handles:
            h.wait()

    def grp_body(g, carry):
        burst(g * DSG, DSG)
        return carry

    lax.fori_loop(0, DFULL, grp_body, 0)
    burst(DFULL * DSG, DCH - DFULL * DSG)
    plsc.subcore_barrier()

    pltpu.sync_copy(ho.at[pl.ds(r0, RPT)], out_o.at[c, pl.ds(r0, RPT)])
    pltpu.sync_copy(hi.at[pl.ds(r0, RPT)], out_i.at[c, pl.ds(r0, RPT)])


@functools.partial(
    pl.kernel,
    mesh=_mesh,
    out_type=jax.ShapeDtypeStruct((NC, NPAD, D), jnp.float32),
    scratch_types=[
        pltpu.VMEM((QC, K), jnp.int32),
        pltpu.VMEM((QC, K), jnp.int32),
        [pltpu.VMEM((K, D), jnp.float32) for _ in range(2)],
        [pltpu.VMEM((K, D), jnp.float32) for _ in range(2)],
        pltpu.VMEM_SHARED((NPAD, D), jnp.float32),
        [pltpu.SemaphoreType.DMA for _ in range(2)],
        [pltpu.SemaphoreType.DMA for _ in range(2)],
    ],
)
def _agg_kernel(featS_hbm, srca_hbm, dstd_hbm, efeat_hbm, zeros2_hbm, out_p,
                src_q, dst_q, rows, ef, hacc, sem_l, sem_s):
    c = lax.axis_index("c")
    s = lax.axis_index("s")
    w = c * NS + s
    r0 = s * RPT

    def load_quarter(q):
        pltpu.sync_copy(srca_hbm.at[w, pl.ds(q * QC, QC), :], src_q)
        pltpu.sync_copy(dstd_hbm.at[w, pl.ds(q * QC, QC), :], dst_q)

    def load_chunk(ch, bs):
        pltpu.async_copy(
            featS_hbm.at[src_q.at[lax.rem(ch, QC)]], rows[bs], sem_l[bs])
        pltpu.async_copy(
            efeat_hbm.at[pl.ds(w * EPW + ch * K, K), :], ef[bs], sem_l[bs])

    def wait_loads(bs):
        pltpu.make_async_copy(
            featS_hbm.at[src_q.at[0]], rows[bs], sem_l[bs]).wait()
        pltpu.make_async_copy(
            efeat_hbm.at[pl.ds(0, K), :], ef[bs], sem_l[bs]).wait()

    def add_ef(bs):
        def rbody(r, cc):
            for rr in range(2):
                for cc16 in range(D // 16):
                    sl = pl.ds(cc16 * 16, 16)
                    plsc.addupdate(rows[bs].at[2 * r + rr, sl],
                                   ef[bs][2 * r + rr, sl])
            return cc
        lax.fori_loop(0, K // 2, rbody, 0)

    def issue_scatters(ch, bs):
        pltpu.async_copy(
            rows[bs], hacc.at[dst_q.at[lax.rem(ch, QC)]], sem_s[bs], add=True)

    def wait_scatters(bs):
        pltpu.make_async_copy(
            rows[bs], hacc.at[dst_q.at[0]], sem_s[bs]).wait()

    pltpu.sync_copy(zeros2_hbm.at[pl.ds(r0, RPT), :], hacc.at[pl.ds(r0, RPT), :])
    plsc.subcore_barrier()

    load_quarter(0)
    load_chunk(0, 0)

    def iter_body(m, carry):
        @pl.when(lax.rem(m, 32) != 0)
        def _():
            wait_scatters(1)
        load_chunk(2 * m + 1, 1)
        wait_loads(0)
        add_ef(0)
        issue_scatters(2 * m, 0)
        wait_scatters(0)

        is_boundary = lax.rem(m, 32) == 31

        @pl.when(is_boundary)
        def _():
            wait_loads(1)
            add_ef(1)
            issue_scatters(2 * m + 1, 1)
            wait_scatters(1)
            load_quarter(lax.div(m, 32) + 1)
            load_chunk(2 * m + 2, 0)

        @pl.when(jnp.logical_not(is_boundary))
        def _():
            load_chunk(2 * m + 2, 0)
            wait_loads(1)
            add_ef(1)
            issue_scatters(2 * m + 1, 1)
        return carry

    lax.fori_loop(0, 124, iter_body, 0)
    wait_scatters(1)
    load_chunk(249, 1)
    wait_loads(0)
    add_ef(0)
    issue_scatters(248, 0)
    wait_scatters(0)
    wait_loads(1)
    add_ef(1)
    issue_scatters(249, 1)
    wait_scatters(1)

    plsc.subcore_barrier()
    pltpu.sync_copy(hacc.at[pl.ds(r0, RPT), :], out_p.at[c, pl.ds(r0, RPT), :])


def _prep_body(deg_ref, feat_ref, out_ref):
    d = deg_ref[:, 0] + deg_ref[:, 1]
    norm = lax.rsqrt(jnp.maximum(d, 1.0))
    out_ref[...] = feat_ref[...] * norm[:, None]


_PB = 1000

_prep_call = pl.pallas_call(
    _prep_body,
    grid=(N // _PB,),
    in_specs=[
        pl.BlockSpec((_PB, NC), lambda i: (i, 0)),
        pl.BlockSpec((_PB, D), lambda i: (i, 0)),
    ],
    out_specs=pl.BlockSpec((_PB, D), lambda i: (i, 0)),
    out_shape=jax.ShapeDtypeStruct((N, D), jnp.float32),
)


def _final_body(p_ref, w_ref, d_ref, b_ref, out_ref):
    h = p_ref[0] + p_ref[1]
    acc = jnp.dot(h, w_ref[...], preferred_element_type=jnp.float32)
    d = d_ref[:, 0] + d_ref[:, 1]
    norm = lax.rsqrt(jnp.maximum(d, 1.0))
    out_ref[...] = acc * norm[:, None] + b_ref[...]


_FB = RPT

_final_call = pl.pallas_call(
    _final_body,
    grid=(NPAD // _FB,),
    in_specs=[
        pl.BlockSpec((NC, _FB, D), lambda i: (0, i, 0)),
        pl.BlockSpec((D, D), lambda i: (0, 0)),
        pl.BlockSpec((_FB, NC), lambda i: (i, 0)),
        pl.BlockSpec((1, D), lambda i: (0, 0)),
    ],
    out_specs=pl.BlockSpec((_FB, D), lambda i: (i, 0)),
    out_shape=jax.ShapeDtypeStruct((NPAD, D), jnp.float32),
)


@jax.jit
def kernel(feat, edge_index, edgeFeat, weight, bias):
    padw = NCHP * K - EPW
    srcp = jnp.pad(edge_index[0].reshape(NW, EPW), ((0, 0), (0, padw)))
    dstp = jnp.pad(edge_index[1].reshape(NW, EPW), ((0, 0), (0, padw)))
    srcq = srcp.reshape(NW, NCHP, K)
    dstq = dstp.reshape(NW, NCHP, K)
    srcd = srcp.reshape(NW, DCH + 3, KD)
    dstd = dstp.reshape(NW, DCH + 3, KD)
    zeros1 = jnp.zeros((NPAD,), jnp.float32)
    zeros2 = jnp.zeros((NPAD, D), jnp.float32)

    deg_o_p, deg_i_p = _deg_kernel(srcd, dstd, zeros1)
    feat_scaled = _prep_call(deg_o_p[:, :N].T, feat)
    partials = _agg_kernel(feat_scaled, srcq, dstq, edgeFeat, zeros2)
    rst = _final_call(partials, weight, deg_i_p.T, bias.reshape(1, D))
    return rst[:N]


# speedup vs baseline: 1.0775x; 1.0775x over previous
import functools

import jax
import jax.numpy as jnp
from jax import lax
from jax.experimental import pallas as pl
from jax.experimental.pallas import tpu as pltpu
from jax.experimental.pallas import tpu_sc as plsc

N = 10000
E = 320000
D = 128

NC = 2
NS = 16
NW = NC * NS

NPAD = 10240
RPT = NPAD // NS
EPW = E // NW

K = 40
NCH = EPW // K
NCHP = 256
QC = 64
NWAVE = NCH // 2

KD = 80
DCH = EPW // KD
DSG = 8
DFULL = DCH // DSG

_mesh = plsc.VectorSubcoreMesh(core_axis_name="c", subcore_axis_name="s")


@functools.partial(
    pl.kernel,
    mesh=_mesh,
    out_type=[
        jax.ShapeDtypeStruct((NC, NPAD), jnp.float32),
        jax.ShapeDtypeStruct((NC, NPAD), jnp.float32),
    ],
    scratch_types=[
        pltpu.VMEM((DCH + 3, KD), jnp.int32),
        pltpu.VMEM((DCH + 3, KD), jnp.int32),
        pltpu.VMEM((KD,), jnp.float32),
        pltpu.VMEM_SHARED((NPAD,), jnp.float32),
        pltpu.VMEM_SHARED((NPAD,), jnp.float32),
        pltpu.SemaphoreType.DMA,
    ],
)
def _deg_kernel(srcd_hbm, dstd_hbm, zeros1_hbm, out_o, out_i,
                src_w, dst_w, ones_v, ho, hi, sem):
    c = lax.axis_index("c")
    s = lax.axis_index("s")
    w = c * NS + s
    r0 = s * RPT

    pltpu.sync_copy(zeros1_hbm.at[pl.ds(r0, RPT)], ho.at[pl.ds(r0, RPT)])
    pltpu.sync_copy(zeros1_hbm.at[pl.ds(r0, RPT)], hi.at[pl.ds(r0, RPT)])
    pltpu.sync_copy(srcd_hbm.at[w], src_w)
    pltpu.sync_copy(dstd_hbm.at[w], dst_w)
    for j in range(KD // 16):
        ones_v[pl.ds(j * 16, 16)] = jnp.full((16,), 1.0, jnp.float32)
    plsc.subcore_barrier()

    def burst(j0, count):
        handles = []
        for b in range(count):
            handles.append(pltpu.async_copy(
                ones_v, ho.at[src_w.at[j0 + b]], sem, add=True))
            handles.append(pltpu.async_copy(
                ones_v, hi.at[dst_w.at[j0 + b]], sem, add=True))
        for h in handles:
            h.wait()

    def grp_body(g, carry):
        burst(g * DSG, DSG)
        return carry

    lax.fori_loop(0, DFULL, grp_body, 0)
    burst(DFULL * DSG, DCH - DFULL * DSG)
    plsc.subcore_barrier()

    pltpu.sync_copy(ho.at[pl.ds(r0, RPT)], out_o.at[c, pl.ds(r0, RPT)])
    pltpu.sync_copy(hi.at[pl.ds(r0, RPT)], out_i.at[c, pl.ds(r0, RPT)])


@functools.partial(
    pl.kernel,
    mesh=_mesh,
    out_type=jax.ShapeDtypeStruct((NC, NPAD, D), jnp.float32),
    scratch_types=[
        pltpu.VMEM((QC, K), jnp.int32),
        pltpu.VMEM((QC, K), jnp.int32),
        [pltpu.VMEM((K, D), jnp.float32) for _ in range(2)],
        [pltpu.VMEM((K, D), jnp.float32) for _ in range(2)],
        pltpu.VMEM_SHARED((NPAD, D), jnp.float32),
        [pltpu.SemaphoreType.DMA for _ in range(2)],
        [pltpu.SemaphoreType.DMA for _ in range(2)],
    ],
)
def _agg_kernel(featS_hbm, srca_hbm, dstd_hbm, efeat_hbm, out_p,
                src_q, dst_q, rows, ef, hacc, sem_l, sem_s):
    c = lax.axis_index("c")
    s = lax.axis_index("s")
    w = c * NS + s
    r0 = s * RPT

    def load_quarter(q):
        pltpu.sync_copy(srca_hbm.at[w, pl.ds(q * QC, QC), :], src_q)
        pltpu.sync_copy(dstd_hbm.at[w, pl.ds(q * QC, QC), :], dst_q)

    def load_chunk(ch, bs):
        pltpu.async_copy(
            featS_hbm.at[src_q.at[lax.rem(ch, QC)]], rows[bs], sem_l[bs])
        pltpu.async_copy(
            efeat_hbm.at[pl.ds(w * EPW + ch * K, K), :], ef[bs], sem_l[bs])

    def wait_loads(bs):
        pltpu.make_async_copy(
            featS_hbm.at[src_q.at[0]], rows[bs], sem_l[bs]).wait()
        pltpu.make_async_copy(
            efeat_hbm.at[pl.ds(0, K), :], ef[bs], sem_l[bs]).wait()

    def issue_scatters(ch, bs):
        pltpu.async_copy(
            rows[bs], hacc.at[dst_q.at[lax.rem(ch, QC)]], sem_s[bs], add=True)
        pltpu.async_copy(
            ef[bs], hacc.at[dst_q.at[lax.rem(ch, QC)]], sem_s[bs], add=True)

    def wait_scatters(bs):
        pltpu.make_async_copy(
            rows[bs], hacc.at[dst_q.at[0]], sem_s[bs]).wait()
        pltpu.make_async_copy(
            ef[bs], hacc.at[dst_q.at[0]], sem_s[bs]).wait()

    def zbody(r, cc):
        for c8 in range(D // 16):
            rows[0][r, pl.ds(c8 * 16, 16)] = jnp.zeros((16,), jnp.float32)
        return cc

    lax.fori_loop(0, K, zbody, 0)
    for t in range(RPT // K):
        pltpu.sync_copy(rows[0], hacc.at[pl.ds(r0 + t * K, K), :])
    plsc.subcore_barrier()

    load_quarter(0)
    load_chunk(0, 0)

    def iter_body(m, carry):
        @pl.when(lax.rem(m, 32) != 0)
        def _():
            wait_scatters(1)
        load_chunk(2 * m + 1, 1)
        wait_loads(0)
        issue_scatters(2 * m, 0)
        wait_scatters(0)

        is_boundary = lax.rem(m, 32) == 31

        @pl.when(is_boundary)
        def _():
            wait_loads(1)
            issue_scatters(2 * m + 1, 1)
            wait_scatters(1)
            load_quarter(lax.div(m, 32) + 1)
            load_chunk(2 * m + 2, 0)

        @pl.when(jnp.logical_not(is_boundary))
        def _():
            load_chunk(2 * m + 2, 0)
            wait_loads(1)
            issue_scatters(2 * m + 1, 1)
        return carry

    lax.fori_loop(0, 124, iter_body, 0)
    wait_scatters(1)
    load_chunk(249, 1)
    wait_loads(0)
    issue_scatters(248, 0)
    wait_scatters(0)
    wait_loads(1)
    issue_scatters(249, 1)
    wait_scatters(1)

    plsc.subcore_barrier()
    pltpu.sync_copy(hacc.at[pl.ds(r0, RPT), :], out_p.at[c, pl.ds(r0, RPT), :])


def _prep_body(deg_ref, feat_ref, out_ref):
    d = deg_ref[:, 0] + deg_ref[:, 1]
    norm = lax.rsqrt(jnp.maximum(d, 1.0))
    out_ref[...] = feat_ref[...] * norm[:, None]


_PB = 1000

_prep_call = pl.pallas_call(
    _prep_body,
    grid=(N // _PB,),
    in_specs=[
        pl.BlockSpec((_PB, NC), lambda i: (i, 0)),
        pl.BlockSpec((_PB, D), lambda i: (i, 0)),
    ],
    out_specs=pl.BlockSpec((_PB, D), lambda i: (i, 0)),
    out_shape=jax.ShapeDtypeStruct((N, D), jnp.float32),
)


def _final_body(p_ref, w_ref, d_ref, b_ref, out_ref):
    h = p_ref[0] + p_ref[1]
    acc = jnp.dot(h, w_ref[...], preferred_element_type=jnp.float32)
    d = d_ref[:, 0] + d_ref[:, 1]
    norm = lax.rsqrt(jnp.maximum(d, 1.0))
    out_ref[...] = acc * norm[:, None] + b_ref[...]


_FB = 400

_final_call = pl.pallas_call(
    _final_body,
    grid=(N // _FB,),
    in_specs=[
        pl.BlockSpec((NC, _FB, D), lambda i: (0, i, 0)),
        pl.BlockSpec((D, D), lambda i: (0, 0)),
        pl.BlockSpec((_FB, NC), lambda i: (i, 0)),
        pl.BlockSpec((1, D), lambda i: (0, 0)),
    ],
    out_specs=pl.BlockSpec((_FB, D), lambda i: (i, 0)),
    out_shape=jax.ShapeDtypeStruct((N, D), jnp.float32),
)


@jax.jit
def kernel(feat, edge_index, edgeFeat, weight, bias):
    padw = NCHP * K - EPW
    srcp = jnp.pad(edge_index[0].reshape(NW, EPW), ((0, 0), (0, padw)))
    dstp = jnp.pad(edge_index[1].reshape(NW, EPW), ((0, 0), (0, padw)))
    srcq = srcp.reshape(NW, NCHP, K)
    dstq = dstp.reshape(NW, NCHP, K)
    srcd = srcp.reshape(NW, DCH + 3, KD)
    dstd = dstp.reshape(NW, DCH + 3, KD)
    zeros1 = jnp.zeros((NPAD,), jnp.float32)

    deg_o_p, deg_i_p = _deg_kernel(srcd, dstd, zeros1)
    feat_scaled = _prep_call(deg_o_p[:, :N].T, feat)
    partials = _agg_kernel(feat_scaled, srcq, dstq, edgeFeat)
    return _final_call(partials, weight, deg_i_p[:, :N].T, bias.reshape(1, D))


# speedup vs baseline: 1.2181x; 1.1305x over previous
import functools

import jax
import jax.numpy as jnp
from jax import lax
from jax.experimental import pallas as pl
from jax.experimental.pallas import tpu as pltpu
from jax.experimental.pallas import tpu_sc as plsc

N = 10000
E = 320000
D = 128

NC = 2
NS = 16
NW = NC * NS

NPAD = 10240
RPT = NPAD // NS
EPW = E // NW

K = 40
NCH = EPW // K
NCHP = 288
QC = 72

KD = 80
DCH = EPW // KD
DSG = 8
DFULL = DCH // DSG

_mesh = plsc.VectorSubcoreMesh(core_axis_name="c", subcore_axis_name="s")


@functools.partial(
    pl.kernel,
    mesh=_mesh,
    out_type=[
        jax.ShapeDtypeStruct((NC, NPAD), jnp.float32),
        jax.ShapeDtypeStruct((NC, NPAD), jnp.float32),
    ],
    scratch_types=[
        pltpu.VMEM((DCH + 3, KD), jnp.int32),
        pltpu.VMEM((DCH + 3, KD), jnp.int32),
        pltpu.VMEM((KD,), jnp.float32),
        pltpu.VMEM_SHARED((NPAD,), jnp.float32),
        pltpu.VMEM_SHARED((NPAD,), jnp.float32),
        pltpu.SemaphoreType.DMA,
    ],
)
def _deg_kernel(srcd_hbm, dstd_hbm, zeros1_hbm, out_o, out_i,
                src_w, dst_w, ones_v, ho, hi, sem):
    c = lax.axis_index("c")
    s = lax.axis_index("s")
    w = c * NS + s
    r0 = s * RPT

    pltpu.sync_copy(zeros1_hbm.at[pl.ds(r0, RPT)], ho.at[pl.ds(r0, RPT)])
    pltpu.sync_copy(zeros1_hbm.at[pl.ds(r0, RPT)], hi.at[pl.ds(r0, RPT)])
    pltpu.sync_copy(srcd_hbm.at[w, pl.ds(0, DCH + 3), :], src_w)
    pltpu.sync_copy(dstd_hbm.at[w, pl.ds(0, DCH + 3), :], dst_w)
    for j in range(KD // 16):
        ones_v[pl.ds(j * 16, 16)] = jnp.full((16,), 1.0, jnp.float32)
    plsc.subcore_barrier()

    def burst(j0, count):
        handles = []
        for b in range(count):
            handles.append(pltpu.async_copy(
                ones_v, ho.at[src_w.at[j0 + b]], sem, add=True))
            handles.append(pltpu.async_copy(
                ones_v, hi.at[dst_w.at[j0 + b]], sem, add=True))
        for h in handles:
            h.wait()

    def grp_body(g, carry):
        burst(g * DSG, DSG)
        return carry

    lax.fori_loop(0, DFULL, grp_body, 0)
    burst(DFULL * DSG, DCH - DFULL * DSG)
    plsc.subcore_barrier()

    pltpu.sync_copy(ho.at[pl.ds(r0, RPT)], out_o.at[c, pl.ds(r0, RPT)])
    pltpu.sync_copy(hi.at[pl.ds(r0, RPT)], out_i.at[c, pl.ds(r0, RPT)])


@functools.partial(
    pl.kernel,
    mesh=_mesh,
    out_type=jax.ShapeDtypeStruct((NC, NPAD, D), jnp.float32),
    scratch_types=[
        pltpu.VMEM((QC, K), jnp.int32),
        pltpu.VMEM((QC, K), jnp.int32),
        [pltpu.VMEM((K, D), jnp.float32) for _ in range(3)],
        [pltpu.VMEM((K, D), jnp.float32) for _ in range(3)],
        pltpu.VMEM_SHARED((NPAD, D), jnp.float32),
        [pltpu.SemaphoreType.DMA for _ in range(3)],
        [pltpu.SemaphoreType.DMA for _ in range(3)],
    ],
)
def _agg_kernel(featS_hbm, srca_hbm, dstd_hbm, efeat_hbm, out_p,
                src_q, dst_q, rows, ef, hacc, sem_l, sem_s):
    c = lax.axis_index("c")
    s = lax.axis_index("s")
    w = c * NS + s
    r0 = s * RPT

    def load_quarter(q):
        pltpu.sync_copy(srca_hbm.at[w, pl.ds(q * QC, QC), :], src_q)
        pltpu.sync_copy(dstd_hbm.at[w, pl.ds(q * QC, QC), :], dst_q)

    def load_chunk(ch, bs):
        pltpu.async_copy(
            featS_hbm.at[src_q.at[lax.rem(ch, QC)]], rows[bs], sem_l[bs])
        pltpu.async_copy(
            efeat_hbm.at[pl.ds(w * EPW + ch * K, K), :], ef[bs], sem_l[bs])

    def wait_loads(bs):
        pltpu.make_async_copy(
            featS_hbm.at[src_q.at[0]], rows[bs], sem_l[bs]).wait()
        pltpu.make_async_copy(
            efeat_hbm.at[pl.ds(0, K), :], ef[bs], sem_l[bs]).wait()

    def issue_scatters(ch, bs):
        pltpu.async_copy(
            rows[bs], hacc.at[dst_q.at[lax.rem(ch, QC)]], sem_s[bs], add=True)
        pltpu.async_copy(
            ef[bs], hacc.at[dst_q.at[lax.rem(ch, QC)]], sem_s[bs], add=True)

    def wait_scatters(bs):
        pltpu.make_async_copy(
            rows[bs], hacc.at[dst_q.at[0]], sem_s[bs]).wait()
        pltpu.make_async_copy(
            ef[bs], hacc.at[dst_q.at[0]], sem_s[bs]).wait()

    def zbody(r, cc):
        for c8 in range(D // 16):
            rows[0][r, pl.ds(c8 * 16, 16)] = jnp.zeros((16,), jnp.float32)
        return cc

    lax.fori_loop(0, K, zbody, 0)
    for t in range(RPT // K):
        pltpu.sync_copy(rows[0], hacc.at[pl.ds(r0 + t * K, K), :])
    plsc.subcore_barrier()

    load_quarter(0)
    load_chunk(0, 0)
    load_chunk(1, 1)

    def iter_body(m, carry):
        boundary = lax.rem(m, 24) == 23

        wait_loads(0)
        issue_scatters(3 * m, 0)

        @pl.when(lax.rem(m, 24) != 0)
        def _():
            wait_scatters(2)
        load_chunk(3 * m + 2, 2)

        wait_loads(1)
        issue_scatters(3 * m + 1, 1)

        @pl.when(jnp.logical_not(boundary))
        def _():
            wait_scatters(0)
            load_chunk(3 * m + 3, 0)

        @pl.when(boundary)
        def _():
            wait_scatters(0)
            wait_loads(2)
            issue_scatters(3 * m + 2, 2)
            wait_scatters(1)
            wait_scatters(2)
            load_quarter(lax.div(m, 24) + 1)
            load_chunk(3 * m + 3, 0)
            load_chunk(3 * m + 4, 1)

        @pl.when(jnp.logical_not(boundary))
        def _():
            wait_loads(2)
            issue_scatters(3 * m + 2, 2)
            wait_scatters(1)

        @pl.when(jnp.logical_and(jnp.logical_not(boundary), m != 82))
        def _():
            load_chunk(3 * m + 4, 1)
        return carry

    lax.fori_loop(0, 83, iter_body, 0)
    wait_loads(0)
    issue_scatters(249, 0)
    wait_scatters(2)
    wait_scatters(0)

    plsc.subcore_barrier()
    pltpu.sync_copy(hacc.at[pl.ds(r0, RPT), :], out_p.at[c, pl.ds(r0, RPT), :])


def _prep_body(deg_ref, feat_ref, out_ref):
    d = deg_ref[:, 0] + deg_ref[:, 1]
    norm = lax.rsqrt(jnp.maximum(d, 1.0))
    out_ref[...] = feat_ref[...] * norm[:, None]


_PB = 1000

_prep_call = pl.pallas_call(
    _prep_body,
    grid=(N // _PB,),
    in_specs=[
        pl.BlockSpec((_PB, NC), lambda i: (i, 0)),
        pl.BlockSpec((_PB, D), lambda i: (i, 0)),
    ],
    out_specs=pl.BlockSpec((_PB, D), lambda i: (i, 0)),
    out_shape=jax.ShapeDtypeStruct((N, D), jnp.float32),
)


def _final_body(p_ref, w_ref, d_ref, b_ref, out_ref):
    h = p_ref[0] + p_ref[1]
    acc = jnp.dot(h, w_ref[...], preferred_element_type=jnp.float32)
    d = d_ref[:, 0] + d_ref[:, 1]
    norm = lax.rsqrt(jnp.maximum(d, 1.0))
    out_ref[...] = acc * norm[:, None] + b_ref[...]


_FB = 400

_final_call = pl.pallas_call(
    _final_body,
    grid=(N // _FB,),
    in_specs=[
        pl.BlockSpec((NC, _FB, D), lambda i: (0, i, 0)),
        pl.BlockSpec((D, D), lambda i: (0, 0)),
        pl.BlockSpec((_FB, NC), lambda i: (i, 0)),
        pl.BlockSpec((1, D), lambda i: (0, 0)),
    ],
    out_specs=pl.BlockSpec((_FB, D), lambda i: (i, 0)),
    out_shape=jax.ShapeDtypeStruct((N, D), jnp.float32),
)


@jax.jit
def kernel(feat, edge_index, edgeFeat, weight, bias):
    padw = NCHP * K - EPW
    srcp = jnp.pad(edge_index[0].reshape(NW, EPW), ((0, 0), (0, padw)))
    dstp = jnp.pad(edge_index[1].reshape(NW, EPW), ((0, 0), (0, padw)))
    srcq = srcp.reshape(NW, NCHP, K)
    dstq = dstp.reshape(NW, NCHP, K)
    srcd = srcp.reshape(NW, NCHP * K // KD, KD)
    dstd = dstp.reshape(NW, NCHP * K // KD, KD)
    zeros1 = jnp.zeros((NPAD,), jnp.float32)

    deg_o_p, deg_i_p = _deg_kernel(srcd, dstd, zeros1)
    feat_scaled = _prep_call(deg_o_p[:, :N].T, feat)
    partials = _agg_kernel(feat_scaled, srcq, dstq, edgeFeat)
    return _final_call(partials, weight, deg_i_p[:, :N].T, bias.reshape(1, D))
